# baseline (speedup 1.0000x reference)
import functools

import jax
import jax.numpy as jnp
from jax import lax
from jax.experimental import pallas as pl
from jax.experimental.pallas import tpu as pltpu
from jax.experimental.pallas import tpu_sc as plsc

NC = 2
NS = 16
NW = NC * NS
CHUNK = 128


def _make_sc_agg(N, D, K):
    agg_rows = ((N + CHUNK + 7) // 8) * 8
    stride = ((-(-agg_rows // NS)) + 7) // 8 * 8
    nfull, rem = stride // CHUNK, stride % CHUNK
    mesh = plsc.VectorSubcoreMesh(
        core_axis_name="c", subcore_axis_name="s",
        num_cores=NC, num_subcores=NS)

    def body(h_hbm, src_hbm, dst_hbm, out_hbm, agg_sh, src_v, dst_v, rows_v,
             rows_w, sem0, sem1, sems0, sems1):
        c = lax.axis_index("c")
        s = lax.axis_index("s")
        wid = s * NC + c

        zv = jnp.zeros((16,), jnp.float32)

        def zrow(i, carry):
            for cc in range(D // 16):
                rows_v[i, pl.ds(cc * 16, 16)] = zv
            return carry

        lax.fori_loop(0, CHUNK, zrow, 0)
        base = jnp.minimum(s * stride, agg_rows - stride)
        for t in range(nfull):
            pltpu.sync_copy(rows_v, agg_sh.at[pl.ds(base + t * CHUNK, CHUNK)])
        if rem:
            pltpu.sync_copy(rows_v.at[pl.ds(0, rem)],
                            agg_sh.at[pl.ds(base + nfull * CHUNK, rem)])
        plsc.subcore_barrier()

        def start_g(j, buf, sm):
            pltpu.async_copy(h_hbm.at[src_v.at[j]], buf, sm)

        def wait_g(buf, sm):
            pltpu.make_async_copy(h_hbm.at[src_v.at[0]], buf, sm).wait()

        def start_s(j, buf, sm):
            pltpu.async_copy(buf, agg_sh.at[dst_v.at[j]], sm, add=True)

        def wait_s(buf, sm):
            pltpu.make_async_copy(buf, agg_sh.at[dst_v.at[0]], sm).wait()

        kh = K // 2
        for half in range(2):
            pltpu.sync_copy(src_hbm.at[wid, pl.ds(half * kh, kh)], src_v)
            pltpu.sync_copy(dst_hbm.at[wid, pl.ds(half * kh, kh)], dst_v)
            start_g(0, rows_v, sem0)
            start_g(1, rows_w, sem1)

            def chunk2(i, carry):
                j = 2 * i
                wait_g(rows_v, sem0)
                start_s(j, rows_v, sems0)
                wait_g(rows_w, sem1)
                start_s(j + 1, rows_w, sems1)
                wait_s(rows_v, sems0)
                start_g(jnp.minimum(j + 2, kh - 1), rows_v, sem0)
                wait_s(rows_w, sems1)
                start_g(jnp.minimum(j + 3, kh - 1), rows_w, sem1)
                return carry

            lax.fori_loop(0, kh // 2, chunk2, 0)
            wait_g(rows_v, sem0)
            wait_g(rows_w, sem1)
        plsc.subcore_barrier()

        pltpu.sync_copy(agg_sh.at[pl.ds(base, stride)],
                        out_hbm.at[c, pl.ds(base, stride)])

    return pl.kernel(
        body,
        out_type=jax.ShapeDtypeStruct((NC, agg_rows, D), jnp.float32),
        mesh=mesh,
        scratch_types=[
            pltpu.VMEM_SHARED((agg_rows, D), jnp.float32),
            pltpu.VMEM((K // 2, CHUNK), jnp.int32),
            pltpu.VMEM((K // 2, CHUNK), jnp.int32),
            pltpu.VMEM((CHUNK, D), jnp.float32),
            pltpu.VMEM((CHUNK, D), jnp.float32),
            pltpu.SemaphoreType.DMA,
            pltpu.SemaphoreType.DMA,
            pltpu.SemaphoreType.DMA,
            pltpu.SemaphoreType.DMA,
        ],
    )


def _mlp_body(h_ref, a_ref, wa_ref, ba_ref, wb_ref, bb_ref, out_ref, *,
              relu_out):
    z = h_ref[...] + a_ref[0] + a_ref[1]
    t = jnp.maximum(
        jnp.dot(z, wa_ref[...], preferred_element_type=jnp.float32)
        + ba_ref[...], 0.0)
    o = (jnp.dot(t, wb_ref[...], preferred_element_type=jnp.float32)
         + bb_ref[...])
    out_ref[...] = jnp.maximum(o, 0.0) if relu_out else o


def _mlp_dual_body(h_ref, a_ref, wa_ref, ba_ref, wb_ref, bb_ref, out_ref,
                   relu_ref):
    z = h_ref[...] + a_ref[0] + a_ref[1]
    t = jnp.maximum(
        jnp.dot(z, wa_ref[...], preferred_element_type=jnp.float32)
        + ba_ref[...], 0.0)
    o = (jnp.dot(t, wb_ref[...], preferred_element_type=jnp.float32)
         + bb_ref[...])
    out_ref[...] = o
    relu_ref[...] = jnp.maximum(o, 0.0)


def _make_mlp(N, D, H, relu_out, dual=False):
    B = 1000
    assert N % B == 0
    grid = (N // B,)
    in_specs = [
        pl.BlockSpec((B, D), lambda i: (i, 0)),
        pl.BlockSpec((NC, B, D), lambda i: (0, i, 0)),
        pl.BlockSpec((D, H), lambda i: (0, 0)),
        pl.BlockSpec((1, H), lambda i: (0, 0)),
        pl.BlockSpec((H, H), lambda i: (0, 0)),
        pl.BlockSpec((1, H), lambda i: (0, 0)),
    ]
    if dual:
        return pl.pallas_call(
            _mlp_dual_body,
            grid=grid,
            in_specs=in_specs,
            out_specs=(pl.BlockSpec((B, H), lambda i: (i, 0)),) * 2,
            out_shape=(jax.ShapeDtypeStruct((N, H), jnp.float32),) * 2,
        )
    return pl.pallas_call(
        functools.partial(_mlp_body, relu_out=relu_out),
        grid=grid,
        in_specs=in_specs,
        out_specs=pl.BlockSpec((B, H), lambda i: (i, 0)),
        out_shape=jax.ShapeDtypeStruct((N, H), jnp.float32),
    )


def kernel(h, edge_index, W1a, b1a, W1b, b1b, W2a, b2a, W2b, b2b):
    N, D = h.shape
    H = W1a.shape[1]
    E = edge_index.shape[1]
    K = -(-(-(-E // (NW * CHUNK))) // 4) * 4
    e_pad = NW * K * CHUNK
    src = jnp.concatenate(
        [edge_index[0], jnp.arange(e_pad - E, dtype=jnp.int32) * 53 % N]
    ).reshape(NW, K, CHUNK)
    dst = jnp.concatenate(
        [edge_index[1], N + jnp.arange(e_pad - E, dtype=jnp.int32) % CHUNK]
    ).reshape(NW, K, CHUNK)

    sc_agg = _make_sc_agg(N, D, K)
    mlp_relu1 = _make_mlp(N, D, H, True)
    mlp_relu = _make_mlp(N, H, H, True)
    mlp_dual = _make_mlp(N, H, H, False, dual=True)
    mlp_plain = _make_mlp(N, H, H, False)

    b1a_ = b1a.reshape(1, H)
    b1b_ = b1b.reshape(1, H)
    b2a_ = b2a.reshape(1, H)
    b2b_ = b2b.reshape(1, H)

    l1 = mlp_relu1(h, sc_agg(h, src, dst), W1a, b1a_, W1b, b1b_)
    l2 = mlp_relu(l1, sc_agg(l1, src, dst), W2a, b2a_, W2b, b2b_)
    l3 = mlp_relu(l2, sc_agg(l2, src, dst), W2a, b2a_, W2b, b2b_)
    l4, r4 = mlp_dual(l3, sc_agg(l3, src, dst), W2a, b2a_, W2b, b2b_)
    l5 = mlp_plain(r4, sc_agg(r4, src, dst), W2a, b2a_, W2b, b2b_)
    return (l5, l4)


# speedup vs baseline: 1.1423x; 1.1423x over previous
import functools

import jax
import jax.numpy as jnp
from jax import lax
from jax.experimental import pallas as pl
from jax.experimental.pallas import tpu as pltpu
from jax.experimental.pallas import tpu_sc as plsc

NC = 2
NS = 16
NW = NC * NS
CHUNK = 128


def _make_sc_agg(N, D, K):
    agg_rows = ((N + CHUNK + 7) // 8) * 8
    stride = ((-(-agg_rows // NS)) + 7) // 8 * 8
    nfull, rem = stride // CHUNK, stride % CHUNK
    mesh = plsc.VectorSubcoreMesh(
        core_axis_name="c", subcore_axis_name="s",
        num_cores=NC, num_subcores=NS)

    def body(h_hbm, src_hbm, dst_hbm, out_hbm, agg_sh, src_v, dst_v, rows_v,
             rows_w, sem0, sem1):
        c = lax.axis_index("c")
        s = lax.axis_index("s")
        wid = s * NC + c

        zv = jnp.zeros((16,), jnp.float32)

        def zrow(i, carry):
            for cc in range(D // 16):
                rows_v[i, pl.ds(cc * 16, 16)] = zv
            return carry

        lax.fori_loop(0, CHUNK, zrow, 0)
        base = jnp.minimum(s * stride, agg_rows - stride)
        for t in range(nfull):
            pltpu.sync_copy(rows_v, agg_sh.at[pl.ds(base + t * CHUNK, CHUNK)])
        if rem:
            pltpu.sync_copy(rows_v.at[pl.ds(0, rem)],
                            agg_sh.at[pl.ds(base + nfull * CHUNK, rem)])
        plsc.subcore_barrier()

        def start_g(j, buf, sm):
            pltpu.async_copy(h_hbm.at[src_v.at[j]], buf, sm)

        def wait_g(buf, sm):
            pltpu.make_async_copy(h_hbm.at[src_v.at[0]], buf, sm).wait()

        kh = K // 2
        for half in range(2):
            pltpu.sync_copy(src_hbm.at[wid, pl.ds(half * kh, kh)], src_v)
            pltpu.sync_copy(dst_hbm.at[wid, pl.ds(half * kh, kh)], dst_v)
            start_g(0, rows_v, sem0)

            def chunk2(i, carry):
                j = 2 * i
                wait_g(rows_v, sem0)
                start_g(j + 1, rows_w, sem1)
                pltpu.sync_copy(rows_v, agg_sh.at[dst_v.at[j]], add=True)
                wait_g(rows_w, sem1)
                start_g(jnp.minimum(j + 2, kh - 1), rows_v, sem0)
                pltpu.sync_copy(rows_w, agg_sh.at[dst_v.at[j + 1]], add=True)
                return carry

            lax.fori_loop(0, kh // 2, chunk2, 0)
            wait_g(rows_v, sem0)
        plsc.subcore_barrier()

        pltpu.sync_copy(agg_sh.at[pl.ds(base, stride)],
                        out_hbm.at[c, pl.ds(base, stride)])

    return pl.kernel(
        body,
        out_type=jax.ShapeDtypeStruct((NC, agg_rows, D), jnp.float32),
        mesh=mesh,
        scratch_types=[
            pltpu.VMEM_SHARED((agg_rows, D), jnp.float32),
            pltpu.VMEM((K // 2, CHUNK), jnp.int32),
            pltpu.VMEM((K // 2, CHUNK), jnp.int32),
            pltpu.VMEM((CHUNK, D), jnp.float32),
            pltpu.VMEM((CHUNK, D), jnp.float32),
            pltpu.SemaphoreType.DMA,
            pltpu.SemaphoreType.DMA,
        ],
    )


def _mlp_body(h_ref, a_ref, wa_ref, ba_ref, wb_ref, bb_ref, out_ref, *,
              relu_out):
    z = h_ref[...] + a_ref[0] + a_ref[1]
    t = jnp.maximum(
        jnp.dot(z, wa_ref[...], preferred_element_type=jnp.float32)
        + ba_ref[...], 0.0)
    o = (jnp.dot(t, wb_ref[...], preferred_element_type=jnp.float32)
         + bb_ref[...])
    out_ref[...] = jnp.maximum(o, 0.0) if relu_out else o


def _mlp_dual_body(h_ref, a_ref, wa_ref, ba_ref, wb_ref, bb_ref, out_ref,
                   relu_ref):
    z = h_ref[...] + a_ref[0] + a_ref[1]
    t = jnp.maximum(
        jnp.dot(z, wa_ref[...], preferred_element_type=jnp.float32)
        + ba_ref[...], 0.0)
    o = (jnp.dot(t, wb_ref[...], preferred_element_type=jnp.float32)
         + bb_ref[...])
    out_ref[...] = o
    relu_ref[...] = jnp.maximum(o, 0.0)


def _make_mlp(N, D, H, relu_out, dual=False):
    B = 2000
    assert N % B == 0
    grid = (N // B,)
    in_specs = [
        pl.BlockSpec((B, D), lambda i: (i, 0)),
        pl.BlockSpec((NC, B, D), lambda i: (0, i, 0)),
        pl.BlockSpec((D, H), lambda i: (0, 0)),
        pl.BlockSpec((1, H), lambda i: (0, 0)),
        pl.BlockSpec((H, H), lambda i: (0, 0)),
        pl.BlockSpec((1, H), lambda i: (0, 0)),
    ]
    if dual:
        return pl.pallas_call(
            _mlp_dual_body,
            grid=grid,
            in_specs=in_specs,
            out_specs=(pl.BlockSpec((B, H), lambda i: (i, 0)),) * 2,
            out_shape=(jax.ShapeDtypeStruct((N, H), jnp.float32),) * 2,
        )
    return pl.pallas_call(
        functools.partial(_mlp_body, relu_out=relu_out),
        grid=grid,
        in_specs=in_specs,
        out_specs=pl.BlockSpec((B, H), lambda i: (i, 0)),
        out_shape=jax.ShapeDtypeStruct((N, H), jnp.float32),
    )


def kernel(h, edge_index, W1a, b1a, W1b, b1b, W2a, b2a, W2b, b2b):
    N, D = h.shape
    H = W1a.shape[1]
    E = edge_index.shape[1]
    K = -(-(-(-E // (NW * CHUNK))) // 4) * 4
    e_pad = NW * K * CHUNK
    src = jnp.concatenate(
        [edge_index[0], jnp.arange(e_pad - E, dtype=jnp.int32) * 53 % N]
    ).reshape(NW, K, CHUNK)
    dst = jnp.concatenate(
        [edge_index[1], N + jnp.arange(e_pad - E, dtype=jnp.int32) % CHUNK]
    ).reshape(NW, K, CHUNK)

    sc_agg = _make_sc_agg(N, D, K)
    mlp_relu1 = _make_mlp(N, D, H, True)
    mlp_relu = _make_mlp(N, H, H, True)
    mlp_dual = _make_mlp(N, H, H, False, dual=True)
    mlp_plain = _make_mlp(N, H, H, False)

    b1a_ = b1a.reshape(1, H)
    b1b_ = b1b.reshape(1, H)
    b2a_ = b2a.reshape(1, H)
    b2b_ = b2b.reshape(1, H)

    l1 = mlp_relu1(h, sc_agg(h, src, dst), W1a, b1a_, W1b, b1b_)
    l2 = mlp_relu(l1, sc_agg(l1, src, dst), W2a, b2a_, W2b, b2b_)
    l3 = mlp_relu(l2, sc_agg(l2, src, dst), W2a, b2a_, W2b, b2b_)
    l4, r4 = mlp_dual(l3, sc_agg(l3, src, dst), W2a, b2a_, W2b, b2b_)
    l5 = mlp_plain(r4, sc_agg(r4, src, dst), W2a, b2a_, W2b, b2b_)
    return (l5, l4)


# speedup vs baseline: 1.1427x; 1.0004x over previous
import functools

import jax
import jax.numpy as jnp
from jax import lax
from jax.experimental import pallas as pl
from jax.experimental.pallas import tpu as pltpu
from jax.experimental.pallas import tpu_sc as plsc

NC = 2
NS = 16
NW = NC * NS
CHUNK = 128


def _make_sc_agg(N, D, K):
    agg_rows = ((N + CHUNK + 7) // 8) * 8
    stride = ((-(-agg_rows // NS)) + 7) // 8 * 8
    nfull, rem = stride // CHUNK, stride % CHUNK
    mesh = plsc.VectorSubcoreMesh(
        core_axis_name="c", subcore_axis_name="s",
        num_cores=NC, num_subcores=NS)

    def body(h_hbm, src_hbm, dst_hbm, out_hbm, agg_sh, src_v, dst_v, rows_v,
             rows_w, sem0, sem1):
        c = lax.axis_index("c")
        s = lax.axis_index("s")
        wid = s * NC + c

        zv = jnp.zeros((16,), jnp.float32)

        def zrow(i, carry):
            for cc in range(D // 16):
                rows_v[i, pl.ds(cc * 16, 16)] = zv
            return carry

        lax.fori_loop(0, CHUNK, zrow, 0)
        base = jnp.minimum(s * stride, agg_rows - stride)
        for t in range(nfull):
            pltpu.sync_copy(rows_v, agg_sh.at[pl.ds(base + t * CHUNK, CHUNK)])
        if rem:
            pltpu.sync_copy(rows_v.at[pl.ds(0, rem)],
                            agg_sh.at[pl.ds(base + nfull * CHUNK, rem)])
        plsc.subcore_barrier()

        def start_g(j, buf, sm):
            pltpu.async_copy(h_hbm.at[src_v.at[j]], buf, sm)

        def wait_g(buf, sm):
            pltpu.make_async_copy(h_hbm.at[src_v.at[0]], buf, sm).wait()

        kh = K // 2
        for half in range(2):
            pltpu.sync_copy(src_hbm.at[wid, pl.ds(half * kh, kh)], src_v)
            pltpu.sync_copy(dst_hbm.at[wid, pl.ds(half * kh, kh)], dst_v)
            start_g(0, rows_v, sem0)

            def chunk4(i, carry):
                j = 4 * i
                for u in range(4):
                    buf_a = rows_v if u % 2 == 0 else rows_w
                    buf_b = rows_w if u % 2 == 0 else rows_v
                    sm_a = sem0 if u % 2 == 0 else sem1
                    sm_b = sem1 if u % 2 == 0 else sem0
                    wait_g(buf_a, sm_a)
                    start_g(jnp.minimum(j + u + 1, kh - 1), buf_b, sm_b)
                    pltpu.sync_copy(buf_a, agg_sh.at[dst_v.at[j + u]],
                                    add=True)
                return carry

            lax.fori_loop(0, kh // 4, chunk4, 0)
            wait_g(rows_v, sem0)
        plsc.subcore_barrier()

        pltpu.sync_copy(agg_sh.at[pl.ds(base, stride)],
                        out_hbm.at[c, pl.ds(base, stride)])

    return pl.kernel(
        body,
        out_type=jax.ShapeDtypeStruct((NC, agg_rows, D), jnp.float32),
        mesh=mesh,
        scratch_types=[
            pltpu.VMEM_SHARED((agg_rows, D), jnp.float32),
            pltpu.VMEM((K // 2, CHUNK), jnp.int32),
            pltpu.VMEM((K // 2, CHUNK), jnp.int32),
            pltpu.VMEM((CHUNK, D), jnp.float32),
            pltpu.VMEM((CHUNK, D), jnp.float32),
            pltpu.SemaphoreType.DMA,
            pltpu.SemaphoreType.DMA,
        ],
    )


def _mlp_body(h_ref, a_ref, wa_ref, ba_ref, wb_ref, bb_ref, out_ref, *,
              relu_out):
    z = h_ref[...] + a_ref[0] + a_ref[1]
    t = jnp.maximum(
        jnp.dot(z, wa_ref[...], preferred_element_type=jnp.float32)
        + ba_ref[...], 0.0)
    o = (jnp.dot(t, wb_ref[...], preferred_element_type=jnp.float32)
         + bb_ref[...])
    out_ref[...] = jnp.maximum(o, 0.0) if relu_out else o


def _mlp_dual_body(h_ref, a_ref, wa_ref, ba_ref, wb_ref, bb_ref, out_ref,
                   relu_ref):
    z = h_ref[...] + a_ref[0] + a_ref[1]
    t = jnp.maximum(
        jnp.dot(z, wa_ref[...], preferred_element_type=jnp.float32)
        + ba_ref[...], 0.0)
    o = (jnp.dot(t, wb_ref[...], preferred_element_type=jnp.float32)
         + bb_ref[...])
    out_ref[...] = o
    relu_ref[...] = jnp.maximum(o, 0.0)


def _make_mlp(N, D, H, relu_out, dual=False):
    B = 2000
    assert N % B == 0
    grid = (N // B,)
    in_specs = [
        pl.BlockSpec((B, D), lambda i: (i, 0)),
        pl.BlockSpec((NC, B, D), lambda i: (0, i, 0)),
        pl.BlockSpec((D, H), lambda i: (0, 0)),
        pl.BlockSpec((1, H), lambda i: (0, 0)),
        pl.BlockSpec((H, H), lambda i: (0, 0)),
        pl.BlockSpec((1, H), lambda i: (0, 0)),
    ]
    if dual:
        return pl.pallas_call(
            _mlp_dual_body,
            grid=grid,
            in_specs=in_specs,
            out_specs=(pl.BlockSpec((B, H), lambda i: (i, 0)),) * 2,
            out_shape=(jax.ShapeDtypeStruct((N, H), jnp.float32),) * 2,
        )
    return pl.pallas_call(
        functools.partial(_mlp_body, relu_out=relu_out),
        grid=grid,
        in_specs=in_specs,
        out_specs=pl.BlockSpec((B, H), lambda i: (i, 0)),
        out_shape=jax.ShapeDtypeStruct((N, H), jnp.float32),
    )


def kernel(h, edge_index, W1a, b1a, W1b, b1b, W2a, b2a, W2b, b2b):
    N, D = h.shape
    H = W1a.shape[1]
    E = edge_index.shape[1]
    K = -(-(-(-E // (NW * CHUNK))) // 8) * 8
    e_pad = NW * K * CHUNK
    src = jnp.concatenate(
        [edge_index[0], jnp.arange(e_pad - E, dtype=jnp.int32) * 53 % N]
    ).reshape(NW, K, CHUNK)
    dst = jnp.concatenate(
        [edge_index[1], N + jnp.arange(e_pad - E, dtype=jnp.int32) % CHUNK]
    ).reshape(NW, K, CHUNK)

    sc_agg = _make_sc_agg(N, D, K)
    mlp_relu1 = _make_mlp(N, D, H, True)
    mlp_relu = _make_mlp(N, H, H, True)
    mlp_dual = _make_mlp(N, H, H, False, dual=True)
    mlp_plain = _make_mlp(N, H, H, False)

    b1a_ = b1a.reshape(1, H)
    b1b_ = b1b.reshape(1, H)
    b2a_ = b2a.reshape(1, H)
    b2b_ = b2b.reshape(1, H)

    l1 = mlp_relu1(h, sc_agg(h, src, dst), W1a, b1a_, W1b, b1b_)
    l2 = mlp_relu(l1, sc_agg(l1, src, dst), W2a, b2a_, W2b, b2b_)
    l3 = mlp_relu(l2, sc_agg(l2, src, dst), W2a, b2a_, W2b, b2b_)
    l4, r4 = mlp_dual(l3, sc_agg(l3, src, dst), W2a, b2a_, W2b, b2b_)
    l5 = mlp_plain(r4, sc_agg(r4, src, dst), W2a, b2a_, W2b, b2b_)
    return (l5, l4)
